# baseline (speedup 1.0000x reference)
import functools

import jax
import jax.numpy as jnp
from jax import lax
from jax.experimental import pallas as pl
from jax.experimental.pallas import tpu as pltpu
from jax.experimental.pallas import tpu_sc as plsc

_NC = 2
_NS = 16
_NW = _NC * _NS
_CHUNK = 128
_RUNROLL = 4
_NBUF = 4


def _emb_body(nb, seq_len, embed, x_hbm, tok_hbm, pos_hbm, out_hbm,
              idx_v, posb, buf0, buf1, buf2, buf3,
              gsem0, gsem1, gsem2, gsem3, wsem0, wsem1, wsem2, wsem3,
              psem):
    bufs = (buf0, buf1, buf2, buf3)
    gsems = (gsem0, gsem1, gsem2, gsem3)
    wsems = (wsem0, wsem1, wsem2, wsem3)
    wid = lax.axis_index("s") * _NC + lax.axis_index("c")
    gb = wid // _NS
    gs = lax.rem(wid, _NS)
    s_base = gs * _CHUNK
    pltpu.sync_copy(x_hbm.at[wid], idx_v)

    def gather(k):
        return pltpu.async_copy(tok_hbm.at[idx_v.at[k]], bufs[k % _NBUF],
                                gsems[k % _NBUF])

    gs_pend = [gather(k) for k in range(_NBUF - 1)]
    pfill = pltpu.async_copy(pos_hbm.at[pl.ds(s_base, _CHUNK)], posb, psem)
    pfill.wait()
    half = _CHUNK // 2
    wbs = [None] * nb
    for k in range(nb):
        buf = bufs[k % _NBUF]
        gs_pend[k].wait()
        if k + _NBUF - 1 < nb:
            if k >= 1:
                for w in wbs[k - 1]:
                    w.wait()
            gs_pend.append(gather(k + _NBUF - 1))

        row0 = (gb * nb + k) * seq_len + s_base
        wbs[k] = []
        for h in range(2):
            r0 = h * half

            @plsc.parallel_loop(r0, r0 + half, step=_RUNROLL, unroll=2)
            def add_rows(i):
                for r in range(_RUNROLL):
                    for j in range(embed // 16):
                        sl = pl.ds(j * 16, 16)
                        plsc.addupdate(buf.at[i + r, sl], posb[i + r, sl])

            wbs[k].append(pltpu.async_copy(
                buf.at[pl.ds(r0, half)],
                out_hbm.at[pl.ds(row0 + r0, half)], wsems[k % _NBUF]))
    for k in range(max(0, nb - _NBUF), nb):
        for w in wbs[k]:
            w.wait()


def kernel(x, token_table, position_table):
    b, s = x.shape
    vocab, embed = token_table.shape
    n = b * s
    ns_groups = s // _CHUNK
    nb_groups = _NW // ns_groups
    nb = b // nb_groups
    assert s % _CHUNK == 0 and _NW % ns_groups == 0 and b % nb_groups == 0
    assert embed % 16 == 0 and _CHUNK % _RUNROLL == 0

    xw = (x.astype(jnp.int32)
          .reshape(nb_groups, nb, ns_groups, _CHUNK)
          .transpose(0, 2, 1, 3)
          .reshape(_NW, nb, _CHUNK))

    mesh = plsc.VectorSubcoreMesh(core_axis_name="c", subcore_axis_name="s")
    body = functools.partial(_emb_body, nb, s, embed)
    out = pl.kernel(
        body,
        mesh=mesh,
        out_type=jax.ShapeDtypeStruct((n, embed), jnp.float32),
        scratch_types=[
            pltpu.VMEM((nb, _CHUNK), jnp.int32),
            pltpu.VMEM((_CHUNK, embed), jnp.float32),
            pltpu.VMEM((_CHUNK, embed), jnp.float32),
            pltpu.VMEM((_CHUNK, embed), jnp.float32),
            pltpu.VMEM((_CHUNK, embed), jnp.float32),
            pltpu.VMEM((_CHUNK, embed), jnp.float32),
            pltpu.SemaphoreType.DMA,
            pltpu.SemaphoreType.DMA,
            pltpu.SemaphoreType.DMA,
            pltpu.SemaphoreType.DMA,
            pltpu.SemaphoreType.DMA,
            pltpu.SemaphoreType.DMA,
            pltpu.SemaphoreType.DMA,
            pltpu.SemaphoreType.DMA,
            pltpu.SemaphoreType.DMA,
        ],
    )(xw, token_table, position_table)
    return out.reshape(b, s, embed)


# speedup vs baseline: 1.0741x; 1.0741x over previous
import functools

import jax
import jax.numpy as jnp
from jax import lax
from jax.experimental import pallas as pl
from jax.experimental.pallas import tpu as pltpu
from jax.experimental.pallas import tpu_sc as plsc

_NC = 2
_NS = 16
_NW = _NC * _NS
_CHUNK = 128
_RUNROLL = 8
_NBUF = 3


def _emb_body(nb, seq_len, embed, x_hbm, tok_hbm, pos_hbm, out_hbm,
              idx_v, posb, buf0, buf1, buf2,
              gsem0, gsem1, gsem2, wsem0, wsem1, wsem2, psem):
    bufs = (buf0, buf1, buf2)
    gsems = (gsem0, gsem1, gsem2)
    wsems = (wsem0, wsem1, wsem2)
    wid = lax.axis_index("s") * _NC + lax.axis_index("c")
    gb = wid // _NS
    gs = lax.rem(wid, _NS)
    s_base = gs * _CHUNK
    pltpu.sync_copy(x_hbm.at[wid], idx_v)

    def gather(k):
        return pltpu.async_copy(tok_hbm.at[idx_v.at[k]], bufs[k % _NBUF],
                                gsems[k % _NBUF])

    gs_pend = [gather(k) for k in range(_NBUF - 1)]
    pfill = pltpu.async_copy(pos_hbm.at[pl.ds(s_base, _CHUNK)], posb, psem)
    pfill.wait()
    wbs = [None] * nb
    for k in range(nb):
        buf = bufs[k % _NBUF]
        gs_pend[k].wait()
        if k + _NBUF - 1 < nb:
            if k >= 1:
                wbs[k - 1].wait()
            gs_pend.append(gather(k + _NBUF - 1))

        @plsc.parallel_loop(0, _CHUNK, step=_RUNROLL, unroll=1)
        def add_rows(i):
            for r in range(_RUNROLL):
                for j in range(embed // 16):
                    sl = pl.ds(j * 16, 16)
                    plsc.addupdate(buf.at[i + r, sl], posb[i + r, sl])

        row0 = (gb * nb + k) * seq_len + s_base
        wbs[k] = pltpu.async_copy(buf, out_hbm.at[pl.ds(row0, _CHUNK)],
                                  wsems[k % _NBUF])
    for k in range(max(0, nb - _NBUF), nb):
        wbs[k].wait()


def kernel(x, token_table, position_table):
    b, s = x.shape
    vocab, embed = token_table.shape
    n = b * s
    ns_groups = s // _CHUNK
    nb_groups = _NW // ns_groups
    nb = b // nb_groups
    assert s % _CHUNK == 0 and _NW % ns_groups == 0 and b % nb_groups == 0
    assert embed % 16 == 0 and _CHUNK % _RUNROLL == 0

    xw = (x.astype(jnp.int32)
          .reshape(nb_groups, nb, ns_groups, _CHUNK)
          .transpose(0, 2, 1, 3)
          .reshape(_NW, nb, _CHUNK))

    mesh = plsc.VectorSubcoreMesh(core_axis_name="c", subcore_axis_name="s")
    body = functools.partial(_emb_body, nb, s, embed)
    out = pl.kernel(
        body,
        mesh=mesh,
        out_type=jax.ShapeDtypeStruct((n, embed), jnp.float32),
        scratch_types=[
            pltpu.VMEM((nb, _CHUNK), jnp.int32),
            pltpu.VMEM((_CHUNK, embed), jnp.float32),
            pltpu.VMEM((_CHUNK, embed), jnp.float32),
            pltpu.VMEM((_CHUNK, embed), jnp.float32),
            pltpu.VMEM((_CHUNK, embed), jnp.float32),
            pltpu.SemaphoreType.DMA,
            pltpu.SemaphoreType.DMA,
            pltpu.SemaphoreType.DMA,
            pltpu.SemaphoreType.DMA,
            pltpu.SemaphoreType.DMA,
            pltpu.SemaphoreType.DMA,
            pltpu.SemaphoreType.DMA,
        ],
    )(xw, token_table, position_table)
    return out.reshape(b, s, embed)


# speedup vs baseline: 1.1626x; 1.0824x over previous
import functools

import jax
import jax.numpy as jnp
from jax import lax
from jax.experimental import pallas as pl
from jax.experimental.pallas import tpu as pltpu
from jax.experimental.pallas import tpu_sc as plsc

_NC = 2
_NS = 16
_NW = _NC * _NS
_CHUNK = 128
_RUNROLL = 4
_NBUF = 4


def _emb_body(nb, seq_len, embed, x_hbm, tok_hbm, pos_hbm, out_hbm,
              idx_v, posb, buf0, buf1, buf2, buf3,
              gsem0, gsem1, gsem2, gsem3, wsem0, wsem1, wsem2, wsem3,
              psem):
    bufs = (buf0, buf1, buf2, buf3)
    gsems = (gsem0, gsem1, gsem2, gsem3)
    wsems = (wsem0, wsem1, wsem2, wsem3)
    wid = lax.axis_index("s") * _NC + lax.axis_index("c")
    gb = wid // _NS
    gs = lax.rem(wid, _NS)
    s_base = gs * _CHUNK
    pltpu.sync_copy(x_hbm.at[wid], idx_v)

    def gather(k):
        return pltpu.async_copy(tok_hbm.at[idx_v.at[k]], bufs[k % _NBUF],
                                gsems[k % _NBUF])

    gs_pend = [gather(k) for k in range(_NBUF)]
    pfill = pltpu.async_copy(pos_hbm.at[pl.ds(s_base, _CHUNK)], posb, psem)
    pfill.wait()
    npair = nb // 2
    wbs = [None] * nb
    for p in range(npair):
        ka, kb = 2 * p, 2 * p + 1
        bufa, bufb = bufs[ka % _NBUF], bufs[kb % _NBUF]
        gs_pend[ka].wait()
        gs_pend[kb].wait()
        if p >= 1 and 2 * p + 2 < nb:
            wbs[2 * p - 2].wait()
            wbs[2 * p - 1].wait()
            gs_pend.append(gather(2 * p + 2))
            gs_pend.append(gather(2 * p + 3))

        @plsc.parallel_loop(0, _CHUNK, step=_RUNROLL, unroll=1)
        def add_rows(i):
            for r in range(_RUNROLL):
                for j in range(embed // 16):
                    sl = pl.ds(j * 16, 16)
                    v = posb[i + r, sl]
                    plsc.addupdate(bufa.at[i + r, sl], v)
                    plsc.addupdate(bufb.at[i + r, sl], v)

        rowa = (gb * nb + ka) * seq_len + s_base
        wbs[ka] = pltpu.async_copy(bufa, out_hbm.at[pl.ds(rowa, _CHUNK)],
                                   wsems[ka % _NBUF])
        rowb = (gb * nb + kb) * seq_len + s_base
        wbs[kb] = pltpu.async_copy(bufb, out_hbm.at[pl.ds(rowb, _CHUNK)],
                                   wsems[kb % _NBUF])
    for k in range(nb - _NBUF, nb):
        wbs[k].wait()


def kernel(x, token_table, position_table):
    b, s = x.shape
    vocab, embed = token_table.shape
    n = b * s
    ns_groups = s // _CHUNK
    nb_groups = _NW // ns_groups
    nb = b // nb_groups
    assert s % _CHUNK == 0 and _NW % ns_groups == 0 and b % nb_groups == 0
    assert embed % 16 == 0 and _CHUNK % _RUNROLL == 0 and nb % 2 == 0

    xw = (x.astype(jnp.int32)
          .reshape(nb_groups, nb, ns_groups, _CHUNK)
          .transpose(0, 2, 1, 3)
          .reshape(_NW, nb, _CHUNK))

    mesh = plsc.VectorSubcoreMesh(core_axis_name="c", subcore_axis_name="s")
    body = functools.partial(_emb_body, nb, s, embed)
    out = pl.kernel(
        body,
        mesh=mesh,
        out_type=jax.ShapeDtypeStruct((n, embed), jnp.float32),
        scratch_types=[
            pltpu.VMEM((nb, _CHUNK), jnp.int32),
            pltpu.VMEM((_CHUNK, embed), jnp.float32),
            pltpu.VMEM((_CHUNK, embed), jnp.float32),
            pltpu.VMEM((_CHUNK, embed), jnp.float32),
            pltpu.VMEM((_CHUNK, embed), jnp.float32),
            pltpu.VMEM((_CHUNK, embed), jnp.float32),
            pltpu.SemaphoreType.DMA,
            pltpu.SemaphoreType.DMA,
            pltpu.SemaphoreType.DMA,
            pltpu.SemaphoreType.DMA,
            pltpu.SemaphoreType.DMA,
            pltpu.SemaphoreType.DMA,
            pltpu.SemaphoreType.DMA,
            pltpu.SemaphoreType.DMA,
            pltpu.SemaphoreType.DMA,
        ],
    )(xw, token_table, position_table)
    return out.reshape(b, s, embed)


# speedup vs baseline: 1.1965x; 1.0292x over previous
import functools

import jax
import jax.numpy as jnp
from jax import lax
from jax.experimental import pallas as pl
from jax.experimental.pallas import tpu as pltpu
from jax.experimental.pallas import tpu_sc as plsc

_NC = 2
_NS = 16
_NW = _NC * _NS
_CHUNK = 128
_RUNROLL = 2
_NBUF = 4


def _emb_body(nb, seq_len, embed, x_hbm, tok_hbm, pos_hbm, out_hbm,
              idx_v, posb, buf0, buf1, buf2, buf3,
              gsem0, gsem1, gsem2, gsem3, wsem0, wsem1, wsem2, wsem3,
              psem):
    bufs = (buf0, buf1, buf2, buf3)
    gsems = (gsem0, gsem1, gsem2, gsem3)
    wsems = (wsem0, wsem1, wsem2, wsem3)
    wid = lax.axis_index("s") * _NC + lax.axis_index("c")
    gb = wid // _NS
    gs = lax.rem(wid, _NS)
    s_base = gs * _CHUNK
    pltpu.sync_copy(x_hbm.at[wid], idx_v)

    def gather(k):
        return pltpu.async_copy(tok_hbm.at[idx_v.at[k]], bufs[k % _NBUF],
                                gsems[k % _NBUF])

    gs_pend = [gather(k) for k in range(_NBUF)]
    pfill = pltpu.async_copy(pos_hbm.at[pl.ds(s_base, _CHUNK)], posb, psem)
    pfill.wait()
    npair = nb // 2
    wbs = [None] * nb
    for p in range(npair):
        ka, kb = 2 * p, 2 * p + 1
        bufa, bufb = bufs[ka % _NBUF], bufs[kb % _NBUF]
        gs_pend[ka].wait()
        gs_pend[kb].wait()
        if p >= 1 and 2 * p + 2 < nb:
            wbs[2 * p - 2].wait()
            wbs[2 * p - 1].wait()
            gs_pend.append(gather(2 * p + 2))
            gs_pend.append(gather(2 * p + 3))

        @plsc.parallel_loop(0, _CHUNK, step=_RUNROLL, unroll=1)
        def add_rows(i):
            for r in range(_RUNROLL):
                for j in range(embed // 16):
                    sl = pl.ds(j * 16, 16)
                    v = posb[i + r, sl]
                    plsc.addupdate(bufa.at[i + r, sl], v)
                    plsc.addupdate(bufb.at[i + r, sl], v)

        rowa = (gb * nb + ka) * seq_len + s_base
        wbs[ka] = pltpu.async_copy(bufa, out_hbm.at[pl.ds(rowa, _CHUNK)],
                                   wsems[ka % _NBUF])
        rowb = (gb * nb + kb) * seq_len + s_base
        wbs[kb] = pltpu.async_copy(bufb, out_hbm.at[pl.ds(rowb, _CHUNK)],
                                   wsems[kb % _NBUF])
    for k in range(nb - _NBUF, nb):
        wbs[k].wait()


def kernel(x, token_table, position_table):
    b, s = x.shape
    vocab, embed = token_table.shape
    n = b * s
    ns_groups = s // _CHUNK
    nb_groups = _NW // ns_groups
    nb = b // nb_groups
    assert s % _CHUNK == 0 and _NW % ns_groups == 0 and b % nb_groups == 0
    assert embed % 16 == 0 and _CHUNK % _RUNROLL == 0 and nb % 2 == 0

    xw = (x.astype(jnp.int32)
          .reshape(nb_groups, nb, ns_groups, _CHUNK)
          .transpose(0, 2, 1, 3)
          .reshape(_NW, nb, _CHUNK))

    mesh = plsc.VectorSubcoreMesh(core_axis_name="c", subcore_axis_name="s")
    body = functools.partial(_emb_body, nb, s, embed)
    out = pl.kernel(
        body,
        mesh=mesh,
        out_type=jax.ShapeDtypeStruct((n, embed), jnp.float32),
        scratch_types=[
            pltpu.VMEM((nb, _CHUNK), jnp.int32),
            pltpu.VMEM((_CHUNK, embed), jnp.float32),
            pltpu.VMEM((_CHUNK, embed), jnp.float32),
            pltpu.VMEM((_CHUNK, embed), jnp.float32),
            pltpu.VMEM((_CHUNK, embed), jnp.float32),
            pltpu.VMEM((_CHUNK, embed), jnp.float32),
            pltpu.SemaphoreType.DMA,
            pltpu.SemaphoreType.DMA,
            pltpu.SemaphoreType.DMA,
            pltpu.SemaphoreType.DMA,
            pltpu.SemaphoreType.DMA,
            pltpu.SemaphoreType.DMA,
            pltpu.SemaphoreType.DMA,
            pltpu.SemaphoreType.DMA,
            pltpu.SemaphoreType.DMA,
        ],
    )(xw, token_table, position_table)
    return out.reshape(b, s, embed)
